# Initial kernel scaffold; baseline (speedup 1.0000x reference)
import jax
import jax.numpy as jnp
from jax.experimental import pallas as pl


def kernel(x, edge_index, edge_type, comp, weight, root, bias):
    raise NotImplementedError("write your pallas kernel here")


# speedup vs baseline: 17.5644x; 17.5644x over previous
import functools

import jax
import jax.numpy as jnp
from jax import lax
from jax.experimental import pallas as pl
from jax.experimental.pallas import tpu as pltpu
from jax.experimental.pallas import tpu_sc as plsc

N = 10000
E = 320000
DIM = 128
R = 20
NBASE = 10

NS = 16
BATCH = 128
NBATCH = 160
CB = 8
NCHUNK = NBATCH // CB
EPT = NBATCH * BATCH
E_PAD = NS * EPT
NPAD = 10240
RPW = NPAD // NS
KPAD = 204800
ZPW = KPAD // NS
LANES = 16

_mesh = plsc.VectorSubcoreMesh(
    core_axis_name="c", subcore_axis_name="s", num_cores=1)


def _wr_body(comp_ref, w2_ref, out_ref):
    out_ref[...] = jnp.dot(comp_ref[...], w2_ref[...],
                           preferred_element_type=jnp.float32)


def _y_body(x_ref, w_ref, y_ref):
    y_ref[0] = jnp.dot(x_ref[...], w_ref[0],
                       preferred_element_type=jnp.float32)


def _inv_body(c_ref, o_ref):
    o_ref[...] = 1.0 / jnp.maximum(c_ref[...], 1.0)


def _final_body(p_ref, yr_ref, b_ref, o_ref):
    o_ref[...] = p_ref[...] + yr_ref[...] + b_ref[...]


@functools.partial(
    pl.kernel,
    mesh=_mesh,
    compiler_params=pltpu.CompilerParams(needs_layout_passes=False),
    out_type=jax.ShapeDtypeStruct((KPAD,), jnp.float32),
    scratch_types=[
        pltpu.VMEM((CB, BATCH), jnp.int32),
        pltpu.VMEM((CB, BATCH), jnp.int32),
        pltpu.VMEM((BATCH,), jnp.float32),
        pltpu.VMEM_SHARED((KPAD,), jnp.float32),
        pltpu.SemaphoreType.DMA,
    ],
)
def _counts_k(dst_hbm, typ_hbm, zk_hbm, out_hbm, dstv, typv, onesv, acc, sem):
    s = lax.axis_index("s")

    pltpu.sync_copy(zk_hbm.at[pl.ds(s * ZPW, ZPW)], acc.at[pl.ds(s * ZPW, ZPW)])

    for k in range(BATCH // LANES):
        onesv[pl.ds(k * LANES, LANES)] = jnp.full((LANES,), 1.0, jnp.float32)

    plsc.subcore_barrier()

    def chunk_body(ci, _):
        pltpu.sync_copy(dst_hbm.at[s, ci], dstv)
        pltpu.sync_copy(typ_hbm.at[s, ci], typv)

        def key_body(b, _):
            for k in range(BATCH // LANES):
                sl = pl.ds(k * LANES, LANES)
                typv[b, sl] = dstv[b, sl] * R + typv[b, sl]
            return 0

        lax.fori_loop(0, CB, key_body, 0)

        def scat_body(b, _):
            pltpu.sync_copy(onesv, acc.at[typv.at[b]], add=True)
            return 0

        lax.fori_loop(0, CB, scat_body, 0)
        return 0

    lax.fori_loop(0, NCHUNK, chunk_body, 0)
    plsc.subcore_barrier()

    pltpu.sync_copy(acc.at[pl.ds(s * ZPW, ZPW)], out_hbm.at[pl.ds(s * ZPW, ZPW)])


@functools.partial(
    pl.kernel,
    mesh=_mesh,
    compiler_params=pltpu.CompilerParams(needs_layout_passes=False),
    out_type=jax.ShapeDtypeStruct((NPAD, DIM), jnp.float32),
    scratch_types=[
        pltpu.VMEM((CB, BATCH), jnp.int32),
        pltpu.VMEM((CB, BATCH), jnp.int32),
        pltpu.VMEM((CB, BATCH), jnp.int32),
        pltpu.VMEM((BATCH, DIM), jnp.float32),
        pltpu.VMEM((BATCH + LANES,), jnp.float32),
        pltpu.VMEM_SHARED((NPAD, DIM), jnp.float32),
        pltpu.SemaphoreType.DMA,
        pltpu.SemaphoreType.DMA,
    ],
)
def _msg_k(src_hbm, typ_hbm, dst_hbm, y_hbm, inv_hbm, zn_hbm, out_hbm,
           srcv, typv, dstv, rows, invv, acc, sem0, sem1):
    s = lax.axis_index("s")

    pltpu.sync_copy(zn_hbm.at[pl.ds(s * RPW, RPW)], acc.at[pl.ds(s * RPW, RPW)])
    plsc.subcore_barrier()

    lane0 = lax.iota(jnp.int32, LANES) == 0

    def chunk_body(ci, _):
        pltpu.sync_copy(src_hbm.at[s, ci], srcv)
        pltpu.sync_copy(typ_hbm.at[s, ci], typv)
        pltpu.sync_copy(dst_hbm.at[s, ci], dstv)

        def idx_body(b, _):
            for k in range(BATCH // LANES):
                sl = pl.ds(k * LANES, LANES)
                a = srcv[b, sl]
                t = typv[b, sl]
                srcv[b, sl] = t * N + a
                typv[b, sl] = dstv[b, sl] * R + t
            return 0

        lax.fori_loop(0, CB, idx_body, 0)

        def batch_body(b, _):
            g = pltpu.async_copy(y_hbm.at[srcv.at[b]], rows, sem0)
            h = pltpu.async_copy(inv_hbm.at[typv.at[b]],
                                 invv.at[pl.ds(0, BATCH)], sem1)
            g.wait()
            h.wait()

            def scale_body(e, _):
                w = invv[pl.ds(e, LANES)]
                sv = jnp.sum(jnp.where(lane0, w, 0.0))
                for k in range(DIM // LANES):
                    sl = pl.ds(k * LANES, LANES)
                    rows[e, sl] = rows[e, sl] * sv
                return 0

            lax.fori_loop(0, BATCH, scale_body, 0)
            pltpu.sync_copy(rows, acc.at[dstv.at[b]], add=True)
            return 0

        lax.fori_loop(0, CB, batch_body, 0)
        return 0

    lax.fori_loop(0, NCHUNK, chunk_body, 0)
    plsc.subcore_barrier()

    pltpu.sync_copy(acc.at[pl.ds(s * RPW, RPW)], out_hbm.at[pl.ds(s * RPW, RPW)])


def kernel(x, edge_index, edge_type, comp, weight, root, bias):
    pad = E_PAD - E
    src = jnp.concatenate(
        [edge_index[0], jnp.zeros((pad,), jnp.int32)]
    ).reshape(NS, NCHUNK, CB, BATCH)
    dst = jnp.concatenate(
        [edge_index[1], jnp.full((pad,), NPAD - 1, jnp.int32)]
    ).reshape(NS, NCHUNK, CB, BATCH)
    typ = jnp.concatenate(
        [edge_type, jnp.zeros((pad,), jnp.int32)]
    ).reshape(NS, NCHUNK, CB, BATCH)

    wr20 = pl.pallas_call(
        _wr_body,
        out_shape=jax.ShapeDtypeStruct((R, DIM * DIM), jnp.float32),
    )(comp, weight.reshape(NBASE, DIM * DIM))
    wr_all = jnp.concatenate(
        [wr20.reshape(R, DIM, DIM), root[None]], axis=0)

    XB = 1000
    y = pl.pallas_call(
        _y_body,
        grid=(N // XB, R + 1),
        in_specs=[
            pl.BlockSpec((XB, DIM), lambda j, r: (j, 0)),
            pl.BlockSpec((1, DIM, DIM), lambda j, r: (r, 0, 0)),
        ],
        out_specs=pl.BlockSpec((1, XB, DIM), lambda j, r: (r, j, 0)),
        out_shape=jax.ShapeDtypeStruct((R + 1, N, DIM), jnp.float32),
    )(x, wr_all)
    y_flat = y.reshape((R + 1) * N, DIM)

    zeros_k = jnp.zeros((KPAD,), jnp.float32)
    counts = _counts_k(dst, typ, zeros_k)

    inv = pl.pallas_call(
        _inv_body,
        out_shape=jax.ShapeDtypeStruct((KPAD // DIM, DIM), jnp.float32),
    )(counts.reshape(KPAD // DIM, DIM))
    inv_flat = inv.reshape(KPAD)

    zeros_n = jnp.zeros((NPAD, DIM), jnp.float32)
    msg = _msg_k(src, typ, dst, y_flat, inv_flat, zeros_n)[:N]

    OB = 1000
    out = pl.pallas_call(
        _final_body,
        grid=(N // OB,),
        in_specs=[
            pl.BlockSpec((OB, DIM), lambda j: (j, 0)),
            pl.BlockSpec((OB, DIM), lambda j: (j, 0)),
            pl.BlockSpec((1, DIM), lambda j: (0, 0)),
        ],
        out_specs=pl.BlockSpec((OB, DIM), lambda j: (j, 0)),
        out_shape=jax.ShapeDtypeStruct((N, DIM), jnp.float32),
    )(msg, y_flat[R * N:(R + 1) * N], bias.reshape(1, DIM))
    return out


# speedup vs baseline: 19.6266x; 1.1174x over previous
import functools

import jax
import jax.numpy as jnp
from jax import lax
from jax.experimental import pallas as pl
from jax.experimental.pallas import tpu as pltpu
from jax.experimental.pallas import tpu_sc as plsc

N = 10000
E = 320000
DIM = 128
R = 20
NBASE = 10

NS = 16
BATCH = 128
NBATCH = 160
CB = 8
NCHUNK = NBATCH // CB
EPT = NBATCH * BATCH
E_PAD = NS * EPT
NPAD = 10240
RPW = NPAD // NS
KPAD = 204800
ZPW = KPAD // NS
LANES = 16

_mesh = plsc.VectorSubcoreMesh(
    core_axis_name="c", subcore_axis_name="s", num_cores=1)


def _wr_body(comp_ref, w2_ref, out_ref):
    out_ref[...] = jnp.dot(comp_ref[...], w2_ref[...],
                           preferred_element_type=jnp.float32)


def _y_body(x_ref, w_ref, y_ref):
    y_ref[0] = jnp.dot(x_ref[...], w_ref[0],
                       preferred_element_type=jnp.float32)


def _inv_body(c_ref, o_ref):
    o_ref[...] = 1.0 / jnp.maximum(c_ref[...], 1.0)


def _final_body(p_ref, yr_ref, b_ref, o_ref):
    o_ref[...] = p_ref[...] + yr_ref[...] + b_ref[...]


@functools.partial(
    pl.kernel,
    mesh=_mesh,
    compiler_params=pltpu.CompilerParams(needs_layout_passes=False),
    out_type=jax.ShapeDtypeStruct((KPAD,), jnp.float32),
    scratch_types=[
        pltpu.VMEM((CB, BATCH), jnp.int32),
        pltpu.VMEM((CB, BATCH), jnp.int32),
        pltpu.VMEM((BATCH,), jnp.float32),
        pltpu.VMEM_SHARED((KPAD,), jnp.float32),
        pltpu.SemaphoreType.DMA,
    ],
)
def _counts_k(dst_hbm, typ_hbm, zk_hbm, out_hbm, dstv, typv, onesv, acc, sem):
    s = lax.axis_index("s")

    pltpu.sync_copy(zk_hbm.at[pl.ds(s * ZPW, ZPW)], acc.at[pl.ds(s * ZPW, ZPW)])

    for k in range(BATCH // LANES):
        onesv[pl.ds(k * LANES, LANES)] = jnp.full((LANES,), 1.0, jnp.float32)

    plsc.subcore_barrier()

    def chunk_body(ci, _):
        pltpu.sync_copy(dst_hbm.at[s, ci], dstv)
        pltpu.sync_copy(typ_hbm.at[s, ci], typv)

        def key_body(b, _):
            for k in range(BATCH // LANES):
                sl = pl.ds(k * LANES, LANES)
                typv[b, sl] = dstv[b, sl] * R + typv[b, sl]
            return 0

        lax.fori_loop(0, CB, key_body, 0)

        def scat_body(b, _):
            pltpu.sync_copy(onesv, acc.at[typv.at[b]], add=True)
            return 0

        lax.fori_loop(0, CB, scat_body, 0)
        return 0

    lax.fori_loop(0, NCHUNK, chunk_body, 0)
    plsc.subcore_barrier()

    pltpu.sync_copy(acc.at[pl.ds(s * ZPW, ZPW)], out_hbm.at[pl.ds(s * ZPW, ZPW)])


@functools.partial(
    pl.kernel,
    mesh=_mesh,
    compiler_params=pltpu.CompilerParams(needs_layout_passes=False),
    out_type=jax.ShapeDtypeStruct((NPAD, DIM), jnp.float32),
    scratch_types=[
        pltpu.VMEM((CB, BATCH), jnp.int32),
        pltpu.VMEM((CB, BATCH), jnp.int32),
        pltpu.VMEM((CB, BATCH), jnp.int32),
        pltpu.VMEM((BATCH, DIM), jnp.float32),
        pltpu.VMEM((BATCH + LANES,), jnp.float32),
        pltpu.VMEM_SHARED((NPAD, DIM), jnp.float32),
        pltpu.SemaphoreType.DMA,
        pltpu.SemaphoreType.DMA,
    ],
)
def _msg_k(src_hbm, typ_hbm, dst_hbm, y_hbm, inv_hbm, zn_hbm, out_hbm,
           srcv, typv, dstv, rows, invv, acc, sem0, sem1):
    s = lax.axis_index("s")

    pltpu.sync_copy(zn_hbm.at[pl.ds(s * RPW, RPW)], acc.at[pl.ds(s * RPW, RPW)])
    plsc.subcore_barrier()

    lane0 = lax.iota(jnp.int32, LANES) == 0

    def chunk_body(ci, _):
        pltpu.sync_copy(src_hbm.at[s, ci], srcv)
        pltpu.sync_copy(typ_hbm.at[s, ci], typv)
        pltpu.sync_copy(dst_hbm.at[s, ci], dstv)

        def idx_body(b, _):
            for k in range(BATCH // LANES):
                sl = pl.ds(k * LANES, LANES)
                a = srcv[b, sl]
                t = typv[b, sl]
                srcv[b, sl] = t * N + a
                typv[b, sl] = dstv[b, sl] * R + t
            return 0

        lax.fori_loop(0, CB, idx_body, 0)

        def batch_body(b, _):
            g = pltpu.async_copy(y_hbm.at[srcv.at[b]], rows, sem0)
            h = pltpu.async_copy(inv_hbm.at[typv.at[b]],
                                 invv.at[pl.ds(0, BATCH)], sem1)
            g.wait()
            h.wait()

            def scale_body(e, _):
                sv = plsc.load_gather(invv, [jnp.full((LANES,), e, jnp.int32)])
                for k in range(DIM // LANES):
                    sl = pl.ds(k * LANES, LANES)
                    rows[e, sl] = rows[e, sl] * sv
                return 0

            lax.fori_loop(0, BATCH, scale_body, 0)
            pltpu.sync_copy(rows, acc.at[dstv.at[b]], add=True)
            return 0

        lax.fori_loop(0, CB, batch_body, 0)
        return 0

    lax.fori_loop(0, NCHUNK, chunk_body, 0)
    plsc.subcore_barrier()

    pltpu.sync_copy(acc.at[pl.ds(s * RPW, RPW)], out_hbm.at[pl.ds(s * RPW, RPW)])


def kernel(x, edge_index, edge_type, comp, weight, root, bias):
    pad = E_PAD - E
    src = jnp.concatenate(
        [edge_index[0], jnp.zeros((pad,), jnp.int32)]
    ).reshape(NS, NCHUNK, CB, BATCH)
    dst = jnp.concatenate(
        [edge_index[1], jnp.full((pad,), NPAD - 1, jnp.int32)]
    ).reshape(NS, NCHUNK, CB, BATCH)
    typ = jnp.concatenate(
        [edge_type, jnp.zeros((pad,), jnp.int32)]
    ).reshape(NS, NCHUNK, CB, BATCH)

    wr20 = pl.pallas_call(
        _wr_body,
        out_shape=jax.ShapeDtypeStruct((R, DIM * DIM), jnp.float32),
    )(comp, weight.reshape(NBASE, DIM * DIM))
    wr_all = jnp.concatenate(
        [wr20.reshape(R, DIM, DIM), root[None]], axis=0)

    XB = 1000
    y = pl.pallas_call(
        _y_body,
        grid=(N // XB, R + 1),
        in_specs=[
            pl.BlockSpec((XB, DIM), lambda j, r: (j, 0)),
            pl.BlockSpec((1, DIM, DIM), lambda j, r: (r, 0, 0)),
        ],
        out_specs=pl.BlockSpec((1, XB, DIM), lambda j, r: (r, j, 0)),
        out_shape=jax.ShapeDtypeStruct((R + 1, N, DIM), jnp.float32),
    )(x, wr_all)
    y_flat = y.reshape((R + 1) * N, DIM)

    zeros_k = jnp.zeros((KPAD,), jnp.float32)
    counts = _counts_k(dst, typ, zeros_k)

    inv = pl.pallas_call(
        _inv_body,
        out_shape=jax.ShapeDtypeStruct((KPAD // DIM, DIM), jnp.float32),
    )(counts.reshape(KPAD // DIM, DIM))
    inv_flat = inv.reshape(KPAD)

    zeros_n = jnp.zeros((NPAD, DIM), jnp.float32)
    msg = _msg_k(src, typ, dst, y_flat, inv_flat, zeros_n)[:N]

    OB = 1000
    out = pl.pallas_call(
        _final_body,
        grid=(N // OB,),
        in_specs=[
            pl.BlockSpec((OB, DIM), lambda j: (j, 0)),
            pl.BlockSpec((OB, DIM), lambda j: (j, 0)),
            pl.BlockSpec((1, DIM), lambda j: (0, 0)),
        ],
        out_specs=pl.BlockSpec((OB, DIM), lambda j: (j, 0)),
        out_shape=jax.ShapeDtypeStruct((N, DIM), jnp.float32),
    )(msg, y_flat[R * N:(R + 1) * N], bias.reshape(1, DIM))
    return out


# speedup vs baseline: 24.2751x; 1.2368x over previous
import functools

import jax
import jax.numpy as jnp
from jax import lax
from jax.experimental import pallas as pl
from jax.experimental.pallas import tpu as pltpu
from jax.experimental.pallas import tpu_sc as plsc

N = 10000
E = 320000
DIM = 128
R = 20
NBASE = 10

NS = 16
BATCH = 128
NBATCH = 160
CB = 8
NCHUNK = NBATCH // CB
EPT = NBATCH * BATCH
E_PAD = NS * EPT
NPAD = 10240
RPW = NPAD // NS
KPAD = 204800
ZPW = KPAD // NS
LANES = 16

_mesh = plsc.VectorSubcoreMesh(
    core_axis_name="c", subcore_axis_name="s", num_cores=1)


def _wr_body(comp_ref, w2_ref, out_ref):
    out_ref[...] = jnp.dot(comp_ref[...], w2_ref[...],
                           preferred_element_type=jnp.float32)


def _y_body(x_ref, w_ref, y_ref):
    y_ref[0] = jnp.dot(x_ref[...], w_ref[0],
                       preferred_element_type=jnp.float32)


def _inv_body(c_ref, o_ref):
    o_ref[...] = 1.0 / jnp.maximum(c_ref[...], 1.0)


def _final_body(p_ref, yr_ref, b_ref, o_ref):
    o_ref[...] = p_ref[...] + yr_ref[...] + b_ref[...]


@functools.partial(
    pl.kernel,
    mesh=_mesh,
    compiler_params=pltpu.CompilerParams(needs_layout_passes=False),
    out_type=jax.ShapeDtypeStruct((KPAD,), jnp.float32),
    scratch_types=[
        pltpu.VMEM((CB, BATCH), jnp.int32),
        pltpu.VMEM((CB, BATCH), jnp.int32),
        pltpu.VMEM((BATCH,), jnp.float32),
        pltpu.VMEM_SHARED((KPAD,), jnp.float32),
        pltpu.SemaphoreType.DMA,
    ],
)
def _counts_k(dst_hbm, typ_hbm, zk_hbm, out_hbm, dstv, typv, onesv, acc, sem):
    s = lax.axis_index("s")

    pltpu.sync_copy(zk_hbm.at[pl.ds(s * ZPW, ZPW)], acc.at[pl.ds(s * ZPW, ZPW)])

    for k in range(BATCH // LANES):
        onesv[pl.ds(k * LANES, LANES)] = jnp.full((LANES,), 1.0, jnp.float32)

    plsc.subcore_barrier()

    def chunk_body(ci, _):
        pltpu.sync_copy(dst_hbm.at[s, ci], dstv)
        pltpu.sync_copy(typ_hbm.at[s, ci], typv)

        def key_body(b, _):
            for k in range(BATCH // LANES):
                sl = pl.ds(k * LANES, LANES)
                typv[b, sl] = dstv[b, sl] * R + typv[b, sl]
            return 0

        lax.fori_loop(0, CB, key_body, 0)

        def scat_body(b, _):
            pltpu.sync_copy(onesv, acc.at[typv.at[b]], add=True)
            return 0

        lax.fori_loop(0, CB, scat_body, 0)
        return 0

    lax.fori_loop(0, NCHUNK, chunk_body, 0)
    plsc.subcore_barrier()

    pltpu.sync_copy(acc.at[pl.ds(s * ZPW, ZPW)], out_hbm.at[pl.ds(s * ZPW, ZPW)])


@functools.partial(
    pl.kernel,
    mesh=_mesh,
    compiler_params=pltpu.CompilerParams(needs_layout_passes=False),
    out_type=jax.ShapeDtypeStruct((NPAD, DIM), jnp.float32),
    scratch_types=[
        pltpu.VMEM((CB, BATCH), jnp.int32),
        pltpu.VMEM((CB, BATCH), jnp.int32),
        pltpu.VMEM((CB, BATCH), jnp.int32),
        pltpu.VMEM((BATCH, DIM), jnp.float32),
        pltpu.VMEM((BATCH, DIM), jnp.float32),
        pltpu.VMEM((BATCH,), jnp.float32),
        pltpu.VMEM((BATCH,), jnp.float32),
        pltpu.VMEM_SHARED((NPAD, DIM), jnp.float32),
        pltpu.SemaphoreType.DMA,
        pltpu.SemaphoreType.DMA,
        pltpu.SemaphoreType.DMA,
        pltpu.SemaphoreType.DMA,
        pltpu.SemaphoreType.DMA,
        pltpu.SemaphoreType.DMA,
    ],
)
def _msg_k(src_hbm, typ_hbm, dst_hbm, y_hbm, inv_hbm, zn_hbm, out_hbm,
           srcv, typv, dstv, rows0, rows1, inv0, inv1, acc,
           gs0, gs1, is0, is1, ss0, ss1):
    s = lax.axis_index("s")

    pltpu.sync_copy(zn_hbm.at[pl.ds(s * RPW, RPW)], acc.at[pl.ds(s * RPW, RPW)])
    plsc.subcore_barrier()

    rows = [rows0, rows1]
    invv = [inv0, inv1]
    gsem = [gs0, gs1]
    isem = [is0, is1]
    ssem = [ss0, ss1]

    def chunk_body(ci, _):
        pltpu.sync_copy(src_hbm.at[s, ci], srcv)
        pltpu.sync_copy(typ_hbm.at[s, ci], typv)
        pltpu.sync_copy(dst_hbm.at[s, ci], dstv)

        def idx_body(b, _):
            for k in range(BATCH // LANES):
                sl = pl.ds(k * LANES, LANES)
                a = srcv[b, sl]
                t = typv[b, sl]
                srcv[b, sl] = t * N + a
                typv[b, sl] = dstv[b, sl] * R + t
            return 0

        lax.fori_loop(0, CB, idx_body, 0)

        def issue(b):
            p = b % 2
            g = pltpu.async_copy(y_hbm.at[srcv.at[b]], rows[p], gsem[p])
            h = pltpu.async_copy(inv_hbm.at[typv.at[b]], invv[p], isem[p])
            return g, h

        def scale(p):
            rbuf, ibuf = rows[p], invv[p]

            def scale_body(e, _):
                sv = plsc.load_gather(ibuf, [jnp.full((LANES,), e, jnp.int32)])
                for k in range(DIM // LANES):
                    sl = pl.ds(k * LANES, LANES)
                    rbuf[e, sl] = rbuf[e, sl] * sv
                return 0

            lax.fori_loop(0, BATCH, scale_body, 0)

        pend = [issue(0), issue(1)]
        scat = [None, None]
        for b in range(CB):
            p = b % 2
            g, h = pend[p]
            g.wait()
            h.wait()
            scale(p)
            scat[p] = pltpu.async_copy(rows[p], acc.at[dstv.at[b]], ssem[p],
                                       add=True)
            scat[p].wait()
            if b + 2 < CB:
                pend[p] = issue(b + 2)
        return 0

    lax.fori_loop(0, NCHUNK, chunk_body, 0)
    plsc.subcore_barrier()

    pltpu.sync_copy(acc.at[pl.ds(s * RPW, RPW)], out_hbm.at[pl.ds(s * RPW, RPW)])


def kernel(x, edge_index, edge_type, comp, weight, root, bias):
    pad = E_PAD - E
    src = jnp.concatenate(
        [edge_index[0], jnp.zeros((pad,), jnp.int32)]
    ).reshape(NS, NCHUNK, CB, BATCH)
    dst = jnp.concatenate(
        [edge_index[1], jnp.full((pad,), NPAD - 1, jnp.int32)]
    ).reshape(NS, NCHUNK, CB, BATCH)
    typ = jnp.concatenate(
        [edge_type, jnp.zeros((pad,), jnp.int32)]
    ).reshape(NS, NCHUNK, CB, BATCH)

    wr20 = pl.pallas_call(
        _wr_body,
        out_shape=jax.ShapeDtypeStruct((R, DIM * DIM), jnp.float32),
    )(comp, weight.reshape(NBASE, DIM * DIM))
    wr_all = jnp.concatenate(
        [wr20.reshape(R, DIM, DIM), root[None]], axis=0)

    XB = 1000
    y = pl.pallas_call(
        _y_body,
        grid=(N // XB, R + 1),
        in_specs=[
            pl.BlockSpec((XB, DIM), lambda j, r: (j, 0)),
            pl.BlockSpec((1, DIM, DIM), lambda j, r: (r, 0, 0)),
        ],
        out_specs=pl.BlockSpec((1, XB, DIM), lambda j, r: (r, j, 0)),
        out_shape=jax.ShapeDtypeStruct((R + 1, N, DIM), jnp.float32),
    )(x, wr_all)
    y_flat = y.reshape((R + 1) * N, DIM)

    zeros_k = jnp.zeros((KPAD,), jnp.float32)
    counts = _counts_k(dst, typ, zeros_k)

    inv = pl.pallas_call(
        _inv_body,
        out_shape=jax.ShapeDtypeStruct((KPAD // DIM, DIM), jnp.float32),
    )(counts.reshape(KPAD // DIM, DIM))
    inv_flat = inv.reshape(KPAD)

    zeros_n = jnp.zeros((NPAD, DIM), jnp.float32)
    msg = _msg_k(src, typ, dst, y_flat, inv_flat, zeros_n)[:N]

    OB = 1000
    out = pl.pallas_call(
        _final_body,
        grid=(N // OB,),
        in_specs=[
            pl.BlockSpec((OB, DIM), lambda j: (j, 0)),
            pl.BlockSpec((OB, DIM), lambda j: (j, 0)),
            pl.BlockSpec((1, DIM), lambda j: (0, 0)),
        ],
        out_specs=pl.BlockSpec((OB, DIM), lambda j: (j, 0)),
        out_shape=jax.ShapeDtypeStruct((N, DIM), jnp.float32),
    )(msg, y_flat[R * N:(R + 1) * N], bias.reshape(1, DIM))
    return out


# speedup vs baseline: 24.5352x; 1.0107x over previous
import functools

import jax
import jax.numpy as jnp
from jax import lax
from jax.experimental import pallas as pl
from jax.experimental.pallas import tpu as pltpu
from jax.experimental.pallas import tpu_sc as plsc

N = 10000
E = 320000
DIM = 128
R = 20
NBASE = 10

NS = 16
BATCH = 128
NBATCH = 160
CB = 8
NCHUNK = NBATCH // CB
EPT = NBATCH * BATCH
E_PAD = NS * EPT
NPAD = 10240
RPW = NPAD // NS
KPAD = 204800
ZPW = KPAD // NS
LANES = 16

_mesh = plsc.VectorSubcoreMesh(
    core_axis_name="c", subcore_axis_name="s", num_cores=1)


def _wr_body(comp_ref, w2_ref, out_ref):
    out_ref[...] = jnp.dot(comp_ref[...], w2_ref[...],
                           preferred_element_type=jnp.float32)


def _y_body(x_ref, w_ref, y_ref):
    y_ref[0] = jnp.dot(x_ref[...], w_ref[0],
                       preferred_element_type=jnp.float32)


def _inv_body(c_ref, o_ref):
    o_ref[...] = 1.0 / jnp.maximum(c_ref[...], 1.0)


def _final_body(p_ref, yr_ref, b_ref, o_ref):
    o_ref[...] = p_ref[...] + yr_ref[...] + b_ref[...]


@functools.partial(
    pl.kernel,
    mesh=_mesh,
    compiler_params=pltpu.CompilerParams(needs_layout_passes=False),
    out_type=jax.ShapeDtypeStruct((KPAD,), jnp.float32),
    scratch_types=[
        pltpu.VMEM((CB, BATCH), jnp.int32),
        pltpu.VMEM((CB, BATCH), jnp.int32),
        pltpu.VMEM((BATCH,), jnp.float32),
        pltpu.VMEM_SHARED((KPAD,), jnp.float32),
        pltpu.SemaphoreType.DMA,
    ],
)
def _counts_k(dst_hbm, typ_hbm, zk_hbm, out_hbm, dstv, typv, onesv, acc, sem):
    s = lax.axis_index("s")

    pltpu.sync_copy(zk_hbm.at[pl.ds(s * ZPW, ZPW)], acc.at[pl.ds(s * ZPW, ZPW)])

    for k in range(BATCH // LANES):
        onesv[pl.ds(k * LANES, LANES)] = jnp.full((LANES,), 1.0, jnp.float32)

    plsc.subcore_barrier()

    def chunk_body(ci, _):
        pltpu.sync_copy(dst_hbm.at[s, ci], dstv)
        pltpu.sync_copy(typ_hbm.at[s, ci], typv)

        def key_body(b, _):
            for k in range(BATCH // LANES):
                sl = pl.ds(k * LANES, LANES)
                typv[b, sl] = dstv[b, sl] * R + typv[b, sl]
            return 0

        lax.fori_loop(0, CB, key_body, 0)

        def scat_body(b, _):
            pltpu.sync_copy(onesv, acc.at[typv.at[b]], add=True)
            return 0

        lax.fori_loop(0, CB, scat_body, 0)
        return 0

    lax.fori_loop(0, NCHUNK, chunk_body, 0)
    plsc.subcore_barrier()

    pltpu.sync_copy(acc.at[pl.ds(s * ZPW, ZPW)], out_hbm.at[pl.ds(s * ZPW, ZPW)])


@functools.partial(
    pl.kernel,
    mesh=_mesh,
    compiler_params=pltpu.CompilerParams(needs_layout_passes=False),
    out_type=jax.ShapeDtypeStruct((NPAD, DIM), jnp.float32),
    scratch_types=[
        pltpu.VMEM((CB, BATCH), jnp.int32),
        pltpu.VMEM((CB, BATCH), jnp.int32),
        pltpu.VMEM((CB, BATCH), jnp.int32),
        pltpu.VMEM((BATCH, DIM), jnp.float32),
        pltpu.VMEM((BATCH, DIM), jnp.float32),
        pltpu.VMEM((BATCH,), jnp.float32),
        pltpu.VMEM((BATCH,), jnp.float32),
        pltpu.VMEM_SHARED((NPAD, DIM), jnp.float32),
        pltpu.SemaphoreType.DMA,
        pltpu.SemaphoreType.DMA,
        pltpu.SemaphoreType.DMA,
        pltpu.SemaphoreType.DMA,
        pltpu.SemaphoreType.DMA,
        pltpu.SemaphoreType.DMA,
    ],
)
def _msg_k(src_hbm, typ_hbm, dst_hbm, y_hbm, inv_hbm, zn_hbm, out_hbm,
           srcv, typv, dstv, rows0, rows1, inv0, inv1, acc,
           gs0, gs1, is0, is1, ss0, ss1):
    s = lax.axis_index("s")

    pltpu.sync_copy(zn_hbm.at[pl.ds(s * RPW, RPW)], acc.at[pl.ds(s * RPW, RPW)])
    plsc.subcore_barrier()

    rows = [rows0, rows1]
    invv = [inv0, inv1]
    gsem = [gs0, gs1]
    isem = [is0, is1]
    ssem = [ss0, ss1]

    def chunk_body(ci, _):
        pltpu.sync_copy(src_hbm.at[s, ci], srcv)
        pltpu.sync_copy(typ_hbm.at[s, ci], typv)
        pltpu.sync_copy(dst_hbm.at[s, ci], dstv)

        def idx_body(b, _):
            for k in range(BATCH // LANES):
                sl = pl.ds(k * LANES, LANES)
                a = srcv[b, sl]
                t = typv[b, sl]
                srcv[b, sl] = t * N + a
                typv[b, sl] = dstv[b, sl] * R + t
            return 0

        lax.fori_loop(0, CB, idx_body, 0)

        def issue(b):
            p = b % 2
            g = pltpu.async_copy(y_hbm.at[srcv.at[b]], rows[p], gsem[p])
            h = pltpu.async_copy(inv_hbm.at[typv.at[b]], invv[p], isem[p])
            return g, h

        def scale(p):
            rbuf, ibuf = rows[p], invv[p]
            UNROLL = 4

            def scale_body(q, _):
                e0 = q * UNROLL
                for u in range(UNROLL):
                    e = e0 + u
                    sv = plsc.load_gather(
                        ibuf, [jnp.full((LANES,), e, jnp.int32)])
                    for k in range(DIM // LANES):
                        sl = pl.ds(k * LANES, LANES)
                        rbuf[e, sl] = rbuf[e, sl] * sv
                return 0

            lax.fori_loop(0, BATCH // UNROLL, scale_body, 0)

        pend = [issue(0), issue(1)]
        scat = [None, None]
        for b in range(CB):
            p = b % 2
            g, h = pend[p]
            g.wait()
            h.wait()
            scale(p)
            scat[p] = pltpu.async_copy(rows[p], acc.at[dstv.at[b]], ssem[p],
                                       add=True)
            scat[p].wait()
            if b + 2 < CB:
                pend[p] = issue(b + 2)
        return 0

    lax.fori_loop(0, NCHUNK, chunk_body, 0)
    plsc.subcore_barrier()

    pltpu.sync_copy(acc.at[pl.ds(s * RPW, RPW)], out_hbm.at[pl.ds(s * RPW, RPW)])


def kernel(x, edge_index, edge_type, comp, weight, root, bias):
    pad = E_PAD - E
    src = jnp.concatenate(
        [edge_index[0], jnp.zeros((pad,), jnp.int32)]
    ).reshape(NS, NCHUNK, CB, BATCH)
    dst = jnp.concatenate(
        [edge_index[1], jnp.full((pad,), NPAD - 1, jnp.int32)]
    ).reshape(NS, NCHUNK, CB, BATCH)
    typ = jnp.concatenate(
        [edge_type, jnp.zeros((pad,), jnp.int32)]
    ).reshape(NS, NCHUNK, CB, BATCH)

    wr20 = pl.pallas_call(
        _wr_body,
        out_shape=jax.ShapeDtypeStruct((R, DIM * DIM), jnp.float32),
    )(comp, weight.reshape(NBASE, DIM * DIM))
    wr_all = jnp.concatenate(
        [wr20.reshape(R, DIM, DIM), root[None]], axis=0)

    XB = 1000
    y = pl.pallas_call(
        _y_body,
        grid=(N // XB, R + 1),
        in_specs=[
            pl.BlockSpec((XB, DIM), lambda j, r: (j, 0)),
            pl.BlockSpec((1, DIM, DIM), lambda j, r: (r, 0, 0)),
        ],
        out_specs=pl.BlockSpec((1, XB, DIM), lambda j, r: (r, j, 0)),
        out_shape=jax.ShapeDtypeStruct((R + 1, N, DIM), jnp.float32),
    )(x, wr_all)
    y_flat = y.reshape((R + 1) * N, DIM)

    zeros_k = jnp.zeros((KPAD,), jnp.float32)
    counts = _counts_k(dst, typ, zeros_k)

    inv = pl.pallas_call(
        _inv_body,
        out_shape=jax.ShapeDtypeStruct((KPAD // DIM, DIM), jnp.float32),
    )(counts.reshape(KPAD // DIM, DIM))
    inv_flat = inv.reshape(KPAD)

    zeros_n = jnp.zeros((NPAD, DIM), jnp.float32)
    msg = _msg_k(src, typ, dst, y_flat, inv_flat, zeros_n)

    OB = 400
    out = pl.pallas_call(
        _final_body,
        grid=(N // OB,),
        in_specs=[
            pl.BlockSpec((OB, DIM), lambda j: (j, 0)),
            pl.BlockSpec((OB, DIM), lambda j: (j, 0)),
            pl.BlockSpec((1, DIM), lambda j: (0, 0)),
        ],
        out_specs=pl.BlockSpec((OB, DIM), lambda j: (j, 0)),
        out_shape=jax.ShapeDtypeStruct((N, DIM), jnp.float32),
    )(msg, y_flat[R * N:(R + 1) * N], bias.reshape(1, DIM))
    return out
